# Initial kernel scaffold; baseline (speedup 1.0000x reference)
import jax
import jax.numpy as jnp
from jax.experimental import pallas as pl


def kernel(keypoints, params):
    raise NotImplementedError("write your pallas kernel here")


# speedup vs baseline: 2.7815x; 2.7815x over previous
import functools

import numpy as np
import jax
import jax.numpy as jnp
from jax.experimental import pallas as pl
from jax.experimental.pallas import tpu as pltpu

_V = 46
_CFG = [(2, 64, 1, False), (64, 64, 1, True), (64, 64, 1, True), (64, 64, 1, True),
        (64, 128, 2, True), (128, 128, 1, True), (128, 128, 1, True),
        (128, 256, 2, True), (256, 256, 1, True), (256, 256, 1, True)]


def _pool_interp_matrix(tq, tout, v):
    s = np.zeros((tq * v, tq), np.float64)
    for t in range(tq):
        s[t * v:(t + 1) * v, t] = 1.0 / v
    pos = (np.arange(tout, dtype=np.float32) * np.float32(tq - 1)
           / np.float32(tout - 1)).astype(np.float64)
    lo = np.floor(pos).astype(np.int64)
    hi = np.clip(lo + 1, 0, tq - 1)
    w = pos - lo
    m = np.zeros((tq, tout), np.float64)
    m[lo, np.arange(tout)] += 1.0 - w
    m[hi, np.arange(tout)] += w
    return (s @ m).astype(np.float32)


_P = _pool_interp_matrix(16, 64, _V)


def _dot(a, b):
    return jnp.dot(a, b, preferred_element_type=jnp.float32)


def _rebar(a):
    return jax.lax.bitcast_convert_type(
        jax.lax.bitcast_convert_type(a, jnp.int32), jnp.float32)


def _to_rows(a, c, t):
    return _rebar(a.reshape(c, t, _V)).reshape(c * t, _V)


def _to_flat(a, c, t):
    return _rebar(a.reshape(c, t, _V)).reshape(c, t * _V)


def _subsample2(a, t):
    c = a.shape[0]
    return a.reshape(c, t // 2, 2 * _V)[:, :, :_V].reshape(c, (t // 2) * _V)


def _make_body(ci, co, it, t, st, has_down, res_kind, first, last):
    tv = t * _V
    to = t // st

    def body(*refs):
        pos = [0]

        def nxt():
            r = refs[pos[0]]
            pos[0] += 1
            return r

        x_ref = nxt()
        pa_r, wa_r, wb_r, wd_r, bd_r, gg_r, gb_r = (nxt() for _ in range(7))
        if has_down:
            dw_r, db_r, dg_r, dbb_r = (nxt() for _ in range(4))
        wt_r, tb_r, tg_r, tbe_r = (nxt() for _ in range(4))
        if res_kind == 'conv':
            rw_r, rb_r, rg_r, rbe_r = (nxt() for _ in range(4))
        if first:
            g0_r, b0_r = nxt(), nxt()
        if last:
            p_r = nxt()
        o_ref = refs[-1]

        x = x_ref[0]
        if first:
            x = x * g0_r[...] + b0_r[...]
        xf = _to_rows(x, ci, t)

        lin = None
        for i in range(3):
            a1 = _to_rows(_dot(wa_r[i], x), it, t)
            a2 = _to_rows(_dot(wb_r[i], x), it, t)
            m = jax.lax.dot_general(a1, a2, (((0,), (0,)), ((), ())),
                                    preferred_element_type=jnp.float32)
            ad = jax.nn.softmax(m * (1.0 / (it * t)), axis=0) + pa_r[i]
            xa = _to_flat(_dot(xf, ad), ci, t)
            c = _dot(wd_r[i], xa)
            lin = c if lin is None else lin + c
        bsum = bd_r[0] + bd_r[1] + bd_r[2]
        out = (lin + bsum) * gg_r[...] + gb_r[...]
        if has_down:
            res = _dot(dw_r[...], x)
            res = (res + db_r[...]) * dg_r[...] + dbb_r[...]
        else:
            res = x
        y = jnp.maximum(out + res, 0.0)

        zpad = jnp.zeros((co, 4 * _V), jnp.float32)
        z = jnp.concatenate([zpad, y, zpad], axis=1)
        tlin = None
        for k in range(9):
            c = _dot(wt_r[k], z[:, k * _V:k * _V + tv])
            tlin = c if tlin is None else tlin + c
        ytc = (tlin + tb_r[...]) * tg_r[...] + tbe_r[...]
        if st == 2:
            ytc = _subsample2(ytc, t)
        if res_kind == 'none':
            res2 = 0.0
        elif res_kind == 'id':
            res2 = x
        else:
            xs = _subsample2(x, t) if st == 2 else x
            res2 = _dot(rw_r[...], xs)
            res2 = (res2 + rb_r[...]) * rg_r[...] + rbe_r[...]
        o = jnp.maximum(ytc + res2, 0.0)
        if last:
            o_ref[0] = jax.lax.dot_general(p_r[...], o, (((0,), (1,)), ((), ())),
                                           preferred_element_type=jnp.float32)
        else:
            o_ref[0] = o

    return body


def _full_spec(shape):
    nd = len(shape)
    return pl.BlockSpec(shape, lambda n: (0,) * nd)


def _run_layer(x, layer, cfg, first, last, g0=None, b0=None):
    ci, co, st, res = cfg
    it = co // 4
    n = x.shape[0]
    t = x.shape[2] // _V
    to = t // st
    gcn, tcn = layer['gcn'], layer['tcn']
    has_down = 'down_w' in gcn
    res_kind = 'none' if not res else ('conv' if 'res' in layer else 'id')

    ins = [x,
           gcn['PA'],
           gcn['wa'].reshape(3, it, ci),
           gcn['wb'].reshape(3, it, ci),
           gcn['wd'].reshape(3, co, ci),
           gcn['bd'].reshape(3, co, 1),
           gcn['bn_g'].reshape(co, 1),
           gcn['bn_b'].reshape(co, 1)]
    if has_down:
        ins += [gcn['down_w'].reshape(co, ci), gcn['down_b'].reshape(co, 1),
                gcn['down_g'].reshape(co, 1), gcn['down_bb'].reshape(co, 1)]
    wt = jnp.transpose(tcn['w'].reshape(co, co, 9), (2, 0, 1))
    ins += [wt, tcn['b'].reshape(co, 1), tcn['g'].reshape(co, 1),
            tcn['be'].reshape(co, 1)]
    if res_kind == 'conv':
        r = layer['res']
        ins += [r['w'].reshape(co, ci), r['b'].reshape(co, 1),
                r['g'].reshape(co, 1), r['be'].reshape(co, 1)]
    if first:
        ins += [g0, b0]
    if last:
        ins += [jnp.asarray(_P)]

    in_specs = [pl.BlockSpec((1, ci, t * _V), lambda n: (n, 0, 0))]
    in_specs += [_full_spec(a.shape) for a in ins[1:]]
    if last:
        out_shape = jax.ShapeDtypeStruct((n, 64, co), jnp.float32)
        out_specs = pl.BlockSpec((1, 64, co), lambda n: (n, 0, 0))
    else:
        out_shape = jax.ShapeDtypeStruct((n, co, to * _V), jnp.float32)
        out_specs = pl.BlockSpec((1, co, to * _V), lambda n: (n, 0, 0))

    body = _make_body(ci, co, it, t, st, has_down, res_kind, first, last)
    return pl.pallas_call(
        body,
        grid=(n,),
        in_specs=in_specs,
        out_specs=out_specs,
        out_shape=out_shape,
        compiler_params=pltpu.CompilerParams(
            dimension_semantics=("arbitrary",)),
    )(*ins)


def kernel(keypoints, params):
    n = keypoints.shape[0]
    x = keypoints[..., 0].reshape(n, 2, 64 * _V)
    g2 = params['data_bn_g'].reshape(_V, 2).T
    b2 = params['data_bn_b'].reshape(_V, 2).T
    g0 = jnp.broadcast_to(g2[:, None, :], (2, 64, _V)).reshape(2, 64 * _V)
    b0 = jnp.broadcast_to(b2[:, None, :], (2, 64, _V)).reshape(2, 64 * _V)
    nlayers = len(_CFG)
    for li, (layer, cfg) in enumerate(zip(params['layers'], _CFG)):
        x = _run_layer(x, layer, cfg, first=(li == 0), last=(li == nlayers - 1),
                       g0=g0 if li == 0 else None, b0=b0 if li == 0 else None)
    return x


# speedup vs baseline: 2.7846x; 1.0011x over previous
import functools

import numpy as np
import jax
import jax.numpy as jnp
from jax.experimental import pallas as pl
from jax.experimental.pallas import tpu as pltpu

_V = 46
_CFG = [(2, 64, 1, False), (64, 64, 1, True), (64, 64, 1, True), (64, 64, 1, True),
        (64, 128, 2, True), (128, 128, 1, True), (128, 128, 1, True),
        (128, 256, 2, True), (256, 256, 1, True), (256, 256, 1, True)]


def _pool_interp_matrix(tq, tout, v):
    s = np.zeros((tq * v, tq), np.float64)
    for t in range(tq):
        s[t * v:(t + 1) * v, t] = 1.0 / v
    pos = (np.arange(tout, dtype=np.float32) * np.float32(tq - 1)
           / np.float32(tout - 1)).astype(np.float64)
    lo = np.floor(pos).astype(np.int64)
    hi = np.clip(lo + 1, 0, tq - 1)
    w = pos - lo
    m = np.zeros((tq, tout), np.float64)
    m[lo, np.arange(tout)] += 1.0 - w
    m[hi, np.arange(tout)] += w
    return (s @ m).astype(np.float32)


_P = _pool_interp_matrix(16, 64, _V)


def _dot(a, b):
    return jnp.dot(a, b, preferred_element_type=jnp.float32)


def _rebar(a):
    return jax.lax.bitcast_convert_type(
        jax.lax.bitcast_convert_type(a, jnp.int32), jnp.float32)


def _to_rows(a, c, t):
    return _rebar(a.reshape(c, t, _V)).reshape(c * t, _V)


def _to_flat(a, c, t):
    return _rebar(a.reshape(c, t, _V)).reshape(c, t * _V)


def _subsample2(a, t):
    c = a.shape[0]
    return a.reshape(c, t // 2, 2 * _V)[:, :, :_V].reshape(c, (t // 2) * _V)


def _make_body(ci, co, it, t, st, has_down, res_kind, first, last):
    tv = t * _V
    to = t // st

    def body(*refs):
        pos = [0]

        def nxt():
            r = refs[pos[0]]
            pos[0] += 1
            return r

        x_ref = nxt()
        pa_r, wa_r, wb_r, wd_r, bd_r, gg_r, gb_r = (nxt() for _ in range(7))
        if has_down:
            dw_r, db_r, dg_r, dbb_r = (nxt() for _ in range(4))
        wt_r, tb_r, tg_r, tbe_r = (nxt() for _ in range(4))
        if res_kind == 'conv':
            rw_r, rb_r, rg_r, rbe_r = (nxt() for _ in range(4))
        if first:
            g0_r, b0_r = nxt(), nxt()
        if last:
            p_r = nxt()
        o_ref = refs[-1]

        x = x_ref[0]
        if first:
            x = x * g0_r[...] + b0_r[...]
        xf = _to_rows(x, ci, t)

        lin = None
        for i in range(3):
            a1 = _to_rows(_dot(wa_r[i], x), it, t)
            a2 = _to_rows(_dot(wb_r[i], x), it, t)
            m = jax.lax.dot_general(a1, a2, (((0,), (0,)), ((), ())),
                                    preferred_element_type=jnp.float32)
            ad = jax.nn.softmax(m * (1.0 / (it * t)), axis=0) + pa_r[i]
            xa = _to_flat(_dot(xf, ad), ci, t)
            c = _dot(wd_r[i], xa)
            lin = c if lin is None else lin + c
        bsum = bd_r[0] + bd_r[1] + bd_r[2]
        out = (lin + bsum) * gg_r[...] + gb_r[...]
        if has_down:
            res = _dot(dw_r[...], x)
            res = (res + db_r[...]) * dg_r[...] + dbb_r[...]
        else:
            res = x
        y = jnp.maximum(out + res, 0.0)

        zpad = jnp.zeros((co, 4 * _V), jnp.float32)
        z = jnp.concatenate([zpad, y, zpad], axis=1)
        tlin = None
        for k in range(9):
            c = _dot(wt_r[k], z[:, k * _V:k * _V + tv])
            tlin = c if tlin is None else tlin + c
        ytc = (tlin + tb_r[...]) * tg_r[...] + tbe_r[...]
        if st == 2:
            ytc = _subsample2(ytc, t)
        if res_kind == 'none':
            res2 = 0.0
        elif res_kind == 'id':
            res2 = x
        else:
            xs = _subsample2(x, t) if st == 2 else x
            res2 = _dot(rw_r[...], xs)
            res2 = (res2 + rb_r[...]) * rg_r[...] + rbe_r[...]
        o = jnp.maximum(ytc + res2, 0.0)
        if last:
            o_ref[0] = jax.lax.dot_general(p_r[...], o, (((0,), (1,)), ((), ())),
                                           preferred_element_type=jnp.float32)
        else:
            o_ref[0] = o

    return body


def _full_spec(shape):
    nd = len(shape)
    return pl.BlockSpec(shape, lambda n: (0,) * nd)


def _run_layer(x, layer, cfg, first, last, g0=None, b0=None):
    ci, co, st, res = cfg
    it = co // 4
    n = x.shape[0]
    t = x.shape[2] // _V
    to = t // st
    gcn, tcn = layer['gcn'], layer['tcn']
    has_down = 'down_w' in gcn
    res_kind = 'none' if not res else ('conv' if 'res' in layer else 'id')

    ins = [x,
           gcn['PA'],
           gcn['wa'].reshape(3, it, ci),
           gcn['wb'].reshape(3, it, ci),
           gcn['wd'].reshape(3, co, ci),
           gcn['bd'].reshape(3, co, 1),
           gcn['bn_g'].reshape(co, 1),
           gcn['bn_b'].reshape(co, 1)]
    if has_down:
        ins += [gcn['down_w'].reshape(co, ci), gcn['down_b'].reshape(co, 1),
                gcn['down_g'].reshape(co, 1), gcn['down_bb'].reshape(co, 1)]
    wt = jnp.transpose(tcn['w'].reshape(co, co, 9), (2, 0, 1))
    ins += [wt, tcn['b'].reshape(co, 1), tcn['g'].reshape(co, 1),
            tcn['be'].reshape(co, 1)]
    if res_kind == 'conv':
        r = layer['res']
        ins += [r['w'].reshape(co, ci), r['b'].reshape(co, 1),
                r['g'].reshape(co, 1), r['be'].reshape(co, 1)]
    if first:
        ins += [g0, b0]
    if last:
        ins += [jnp.asarray(_P)]

    in_specs = [pl.BlockSpec((1, ci, t * _V), lambda n: (n, 0, 0))]
    in_specs += [_full_spec(a.shape) for a in ins[1:]]
    if last:
        out_shape = jax.ShapeDtypeStruct((n, 64, co), jnp.float32)
        out_specs = pl.BlockSpec((1, 64, co), lambda n: (n, 0, 0))
    else:
        out_shape = jax.ShapeDtypeStruct((n, co, to * _V), jnp.float32)
        out_specs = pl.BlockSpec((1, co, to * _V), lambda n: (n, 0, 0))

    body = _make_body(ci, co, it, t, st, has_down, res_kind, first, last)
    return pl.pallas_call(
        body,
        grid=(n,),
        in_specs=in_specs,
        out_specs=out_specs,
        out_shape=out_shape,
        compiler_params=pltpu.CompilerParams(
            dimension_semantics=("parallel",)),
    )(*ins)


def kernel(keypoints, params):
    n = keypoints.shape[0]
    x = keypoints[..., 0].reshape(n, 2, 64 * _V)
    g2 = params['data_bn_g'].reshape(_V, 2).T
    b2 = params['data_bn_b'].reshape(_V, 2).T
    g0 = jnp.broadcast_to(g2[:, None, :], (2, 64, _V)).reshape(2, 64 * _V)
    b0 = jnp.broadcast_to(b2[:, None, :], (2, 64, _V)).reshape(2, 64 * _V)
    nlayers = len(_CFG)
    for li, (layer, cfg) in enumerate(zip(params['layers'], _CFG)):
        x = _run_layer(x, layer, cfg, first=(li == 0), last=(li == nlayers - 1),
                       g0=g0 if li == 0 else None, b0=b0 if li == 0 else None)
    return x


# speedup vs baseline: 3.0277x; 1.0873x over previous
import functools

import numpy as np
import jax
import jax.numpy as jnp
from jax.experimental import pallas as pl
from jax.experimental.pallas import tpu as pltpu

_V = 46
_VP = 64
_CFG = [(2, 64, 1, False), (64, 64, 1, True), (64, 64, 1, True), (64, 64, 1, True),
        (64, 128, 2, True), (128, 128, 1, True), (128, 128, 1, True),
        (128, 256, 2, True), (256, 256, 1, True), (256, 256, 1, True)]


def _pool_interp_matrix(tq, tout, v, vp):
    s = np.zeros((tq * vp, tq), np.float64)
    for t in range(tq):
        s[t * vp:t * vp + v, t] = 1.0 / v
    pos = (np.arange(tout, dtype=np.float32) * np.float32(tq - 1)
           / np.float32(tout - 1)).astype(np.float64)
    lo = np.floor(pos).astype(np.int64)
    hi = np.clip(lo + 1, 0, tq - 1)
    w = pos - lo
    m = np.zeros((tq, tout), np.float64)
    m[lo, np.arange(tout)] += 1.0 - w
    m[hi, np.arange(tout)] += w
    return (s @ m).astype(np.float32)


_P = _pool_interp_matrix(16, 64, _V, _VP)


def _dot(a, b):
    return jnp.dot(a, b, preferred_element_type=jnp.float32)


def _rebar(a):
    return jax.lax.bitcast_convert_type(
        jax.lax.bitcast_convert_type(a, jnp.int32), jnp.float32)


def _to_rows(a, c, t):
    return _rebar(a.reshape(c, t, _VP)).reshape(c * t, _VP)


def _to_flat(a, c, t):
    return _rebar(a.reshape(c, t, _VP)).reshape(c, t * _VP)


def _subsample2(a, t):
    c = a.shape[0]
    return a.reshape(c, t // 2, 2 * _VP)[:, :, :_VP].reshape(c, (t // 2) * _VP)


def _make_body(ci, co, it, t, st, has_down, res_kind, first, last):
    tv = t * _VP
    to = t // st

    def body(*refs):
        pos = [0]

        def nxt():
            r = refs[pos[0]]
            pos[0] += 1
            return r

        x_ref = nxt()
        pa_r, wa_r, wb_r, wd_r, bd_r, gg_r, gb_r = (nxt() for _ in range(7))
        if has_down:
            dw_r, db_r, dg_r, dbb_r = (nxt() for _ in range(4))
        wt_r, tb_r, tg_r, tbe_r = (nxt() for _ in range(4))
        if res_kind == 'conv':
            rw_r, rb_r, rg_r, rbe_r = (nxt() for _ in range(4))
        if first:
            g0_r, b0_r = nxt(), nxt()
        if last:
            p_r = nxt()
        o_ref = refs[-1]

        x = x_ref[0]
        if first:
            x = x * g0_r[...] + b0_r[...]
        xf = _to_rows(x, ci, t)

        lin = None
        for i in range(3):
            a1 = _to_rows(_dot(wa_r[i], x), it, t)
            a2 = _to_rows(_dot(wb_r[i], x), it, t)
            m = jax.lax.dot_general(a1, a2, (((0,), (0,)), ((), ())),
                                    preferred_element_type=jnp.float32)
            row = jax.lax.broadcasted_iota(jnp.int32, (_VP, _VP), 0)
            col = jax.lax.broadcasted_iota(jnp.int32, (_VP, _VP), 1)
            m = jnp.where(row < _V, m * (1.0 / (it * t)), -1e30)
            ad = jnp.where(col < _V, jax.nn.softmax(m, axis=0) + pa_r[i], 0.0)
            xa = _to_flat(_dot(xf, ad), ci, t)
            c = _dot(wd_r[i], xa)
            lin = c if lin is None else lin + c
        bsum = bd_r[0] + bd_r[1] + bd_r[2]
        out = (lin + bsum) * gg_r[...] + gb_r[...]
        if has_down:
            res = _dot(dw_r[...], x)
            res = (res + db_r[...]) * dg_r[...] + dbb_r[...]
        else:
            res = x
        y = jnp.maximum(out + res, 0.0)

        zpad = jnp.zeros((co, 4 * _VP), jnp.float32)
        z = jnp.concatenate([zpad, y, zpad], axis=1)
        tlin = None
        for k in range(9):
            c = _dot(wt_r[k], z[:, k * _VP:k * _VP + tv])
            tlin = c if tlin is None else tlin + c
        ytc = (tlin + tb_r[...]) * tg_r[...] + tbe_r[...]
        if st == 2:
            ytc = _subsample2(ytc, t)
        if res_kind == 'none':
            res2 = 0.0
        elif res_kind == 'id':
            res2 = x
        else:
            xs = _subsample2(x, t) if st == 2 else x
            res2 = _dot(rw_r[...], xs)
            res2 = (res2 + rb_r[...]) * rg_r[...] + rbe_r[...]
        o = jnp.maximum(ytc + res2, 0.0)
        if last:
            o_ref[0] = jax.lax.dot_general(p_r[...], o, (((0,), (1,)), ((), ())),
                                           preferred_element_type=jnp.float32)
        else:
            o_ref[0] = o

    return body


def _full_spec(shape):
    nd = len(shape)
    return pl.BlockSpec(shape, lambda n: (0,) * nd)


def _run_layer(x, layer, cfg, first, last, g0=None, b0=None):
    ci, co, st, res = cfg
    it = co // 4
    n = x.shape[0]
    t = x.shape[2] // _VP
    to = t // st
    gcn, tcn = layer['gcn'], layer['tcn']
    has_down = 'down_w' in gcn
    res_kind = 'none' if not res else ('conv' if 'res' in layer else 'id')

    pa = jnp.zeros((3, _VP, _VP), jnp.float32).at[:, :_V, :_V].set(gcn['PA'])
    ins = [x,
           pa,
           gcn['wa'].reshape(3, it, ci),
           gcn['wb'].reshape(3, it, ci),
           gcn['wd'].reshape(3, co, ci),
           gcn['bd'].reshape(3, co, 1),
           gcn['bn_g'].reshape(co, 1),
           gcn['bn_b'].reshape(co, 1)]
    if has_down:
        ins += [gcn['down_w'].reshape(co, ci), gcn['down_b'].reshape(co, 1),
                gcn['down_g'].reshape(co, 1), gcn['down_bb'].reshape(co, 1)]
    wt = jnp.transpose(tcn['w'].reshape(co, co, 9), (2, 0, 1))
    ins += [wt, tcn['b'].reshape(co, 1), tcn['g'].reshape(co, 1),
            tcn['be'].reshape(co, 1)]
    if res_kind == 'conv':
        r = layer['res']
        ins += [r['w'].reshape(co, ci), r['b'].reshape(co, 1),
                r['g'].reshape(co, 1), r['be'].reshape(co, 1)]
    if first:
        ins += [g0, b0]
    if last:
        ins += [jnp.asarray(_P)]

    in_specs = [pl.BlockSpec((1, ci, t * _VP), lambda n: (n, 0, 0))]
    in_specs += [_full_spec(a.shape) for a in ins[1:]]
    if last:
        out_shape = jax.ShapeDtypeStruct((n, 64, co), jnp.float32)
        out_specs = pl.BlockSpec((1, 64, co), lambda n: (n, 0, 0))
    else:
        out_shape = jax.ShapeDtypeStruct((n, co, to * _VP), jnp.float32)
        out_specs = pl.BlockSpec((1, co, to * _VP), lambda n: (n, 0, 0))

    body = _make_body(ci, co, it, t, st, has_down, res_kind, first, last)
    return pl.pallas_call(
        body,
        grid=(n,),
        in_specs=in_specs,
        out_specs=out_specs,
        out_shape=out_shape,
        compiler_params=pltpu.CompilerParams(
            dimension_semantics=("parallel",)),
    )(*ins)


def kernel(keypoints, params):
    n = keypoints.shape[0]
    xr = keypoints[..., 0]
    x = jnp.pad(xr, ((0, 0), (0, 0), (0, 0), (0, _VP - _V)))
    x = x.reshape(n, 2, 64 * _VP)
    g2 = params['data_bn_g'].reshape(_V, 2).T
    b2 = params['data_bn_b'].reshape(_V, 2).T
    g0 = jnp.pad(jnp.broadcast_to(g2[:, None, :], (2, 64, _V)),
                 ((0, 0), (0, 0), (0, _VP - _V))).reshape(2, 64 * _VP)
    b0 = jnp.pad(jnp.broadcast_to(b2[:, None, :], (2, 64, _V)),
                 ((0, 0), (0, 0), (0, _VP - _V))).reshape(2, 64 * _VP)
    nlayers = len(_CFG)
    for li, (layer, cfg) in enumerate(zip(params['layers'], _CFG)):
        x = _run_layer(x, layer, cfg, first=(li == 0), last=(li == nlayers - 1),
                       g0=g0 if li == 0 else None, b0=b0 if li == 0 else None)
    return x
